# baseline (speedup 1.0000x reference)
import functools

import jax
import jax.numpy as jnp
from jax import lax
from jax.experimental import pallas as pl
from jax.experimental.pallas import tpu as pltpu
from jax.experimental.pallas import tpu_sc as plsc

N = 10000
E = 320000
HIDDEN = 128

NC = 2
NS = 16
L = 16
NW = NC * NS
EPW = E // NW
CHUNK = 80
NCHUNK = EPW // CHUNK
NPAD = 10240
RPT = NPAD // NS


def _pre_body(feat_ref, w_ref, asrc_ref, adst_ref, h_ref, es_ref, ed_ref, m_ref):
    h = jnp.dot(feat_ref[...], w_ref[...], preferred_element_type=jnp.float32)
    h_ref[...] = h
    es = jnp.dot(h, asrc_ref[0, :], preferred_element_type=jnp.float32)
    ed = jnp.dot(h, adst_ref[0, :], preferred_element_type=jnp.float32)
    es_ref[0, :] = es
    ed_ref[0, :] = ed
    t = jnp.max(es) + jnp.max(ed)
    m_ref[0, 0] = jnp.where(t >= 0.0, t, 0.2 * t)


def _tc_pre(feat, W, a_src, a_dst):
    return pl.pallas_call(
        _pre_body,
        out_shape=(
            jax.ShapeDtypeStruct((N, HIDDEN), jnp.float32),
            jax.ShapeDtypeStruct((1, N), jnp.float32),
            jax.ShapeDtypeStruct((1, N), jnp.float32),
            jax.ShapeDtypeStruct((1, 1), jnp.float32),
        ),
        out_specs=(
            pl.BlockSpec((N, HIDDEN), lambda: (0, 0)),
            pl.BlockSpec((1, N), lambda: (0, 0)),
            pl.BlockSpec((1, N), lambda: (0, 0)),
            pl.BlockSpec(memory_space=pltpu.SMEM),
        ),
    )(feat, W, a_src.reshape(1, HIDDEN), a_dst.reshape(1, HIDDEN))


def _sc_body(h_hbm, es_hbm, ed_hbm, m_hbm, src_hbm, dst_hbm,
             acc_out, den_out,
             den_v, rows_v, srcb, dstb, dst_scat, esb, edb, m_v,
             acc_sh, es_sp, ed_sp,
             sem_idx, sem_rows, sem_e, sem_scat):
    cid = lax.axis_index("c")
    sid = lax.axis_index("s")
    wid = sid * NC + cid
    base = wid * EPW

    pltpu.sync_copy(m_hbm, m_v)

    @pl.when(sid == 0)
    def _():
        pltpu.sync_copy(es_hbm, es_sp)
        pltpu.sync_copy(ed_hbm, ed_sp)

    def _zrow(j, _):
        for k in range(HIDDEN // L):
            rows_v[0][j, pl.ds(k * L, L)] = jnp.zeros((L,), jnp.float32)
        return 0
    lax.fori_loop(0, CHUNK, _zrow, 0)

    def _zden(j, _):
        den_v[pl.ds(j * L, L)] = jnp.zeros((L,), jnp.float32)
        return 0
    lax.fori_loop(0, N // L, _zden, 0)

    for t in range(RPT // CHUNK):
        pltpu.sync_copy(rows_v[0], acc_sh.at[pl.ds(sid * RPT + t * CHUNK, CHUNK)])
    plsc.subcore_barrier()

    def _copy_idx(i, b):
        off = base + i * CHUNK
        pltpu.async_copy(src_hbm.at[pl.ds(off, CHUNK)], srcb[b], sem_idx[b])
        pltpu.async_copy(dst_hbm.at[pl.ds(off, CHUNK)], dstb[b], sem_idx[b])

    def _wait_idx(b):
        pltpu.make_async_copy(src_hbm.at[pl.ds(0, CHUNK)], srcb[b], sem_idx[b]).wait()
        pltpu.make_async_copy(dst_hbm.at[pl.ds(0, CHUNK)], dstb[b], sem_idx[b]).wait()

    def _start_fetch(b):
        _wait_idx(b)
        pltpu.async_copy(h_hbm.at[srcb[b]], rows_v[b], sem_rows[b])
        pltpu.async_copy(es_sp.at[srcb[b]], esb[b], sem_e[b])
        pltpu.async_copy(ed_sp.at[dstb[b]], edb[b], sem_e[b])

    def _wait_fetch(b):
        pltpu.make_async_copy(h_hbm.at[srcb[b]], rows_v[b], sem_rows[b]).wait()
        pltpu.make_async_copy(es_sp.at[srcb[b]], esb[b], sem_e[b]).wait()
        pltpu.make_async_copy(ed_sp.at[dstb[b]], edb[b], sem_e[b]).wait()

    def _wait_scat(b):
        pltpu.make_async_copy(rows_v[b], acc_sh.at[dst_scat[b]], sem_scat[b]).wait()

    def _snap_dst(b):
        for g in range(CHUNK // L):
            dst_scat[b][pl.ds(g * L, L)] = dstb[b][pl.ds(g * L, L)]

    def _compute(b):
        for g in range(CHUNK // L):
            s = esb[b][pl.ds(g * L, L)] + edb[b][pl.ds(g * L, L)]
            e = jnp.where(s >= 0.0, s, 0.2 * s)
            au = jnp.exp(e - m_v[...])
            plsc.addupdate_scatter(den_v, [dst_scat[b][pl.ds(g * L, L)]], au)
            for jl in range(L):
                ab = jnp.take_along_axis(au, jnp.full((L,), jl, jnp.int32), axis=0)
                j = g * L + jl
                for k in range(HIDDEN // L):
                    rows_v[b][j, pl.ds(k * L, L)] = rows_v[b][j, pl.ds(k * L, L)] * ab

    def _start_scat(b):
        pltpu.async_copy(rows_v[b], acc_sh.at[dst_scat[b]], add=True, sem=sem_scat[b])

    _copy_idx(0, 0)
    _start_fetch(0)
    _copy_idx(1, 1)

    def _step(i, b, first, last):
        if first:
            @pl.when(i >= 1)
            def _():
                _wait_scat(1 - b)
        else:
            _wait_scat(1 - b)
        if not last:
            _start_fetch(1 - b)
        _wait_fetch(b)
        _snap_dst(b)
        if not last:
            @pl.when(i + 2 < NCHUNK)
            def _():
                _copy_idx(i + 2, b)
        _compute(b)
        _start_scat(b)

    def _pair(k, _):
        _step(2 * k, 0, first=True, last=False)
        _step(2 * k + 1, 1, first=False, last=False)
        return 0

    lax.fori_loop(0, (NCHUNK - 1) // 2, _pair, 0)
    _step(NCHUNK - 1, (NCHUNK - 1) % 2, first=False, last=True)
    _wait_scat((NCHUNK - 1) % 2)

    pltpu.sync_copy(den_v, den_out.at[wid, 0])
    plsc.subcore_barrier()
    pltpu.sync_copy(acc_sh.at[pl.ds(sid * RPT, RPT)],
                    acc_out.at[cid, pl.ds(sid * RPT, RPT)])


def _sc_agg(h, es, ed, m, src, dst):
    fn = pl.kernel(
        _sc_body,
        mesh=plsc.VectorSubcoreMesh(core_axis_name="c", subcore_axis_name="s"),
        compiler_params=pltpu.CompilerParams(needs_layout_passes=False),
        out_type=(
            jax.ShapeDtypeStruct((NC, NPAD, HIDDEN), jnp.float32),
            jax.ShapeDtypeStruct((NW, 1, N), jnp.float32),
        ),
        scratch_types=[
            pltpu.VMEM((N,), jnp.float32),
            [pltpu.VMEM((CHUNK, HIDDEN), jnp.float32)] * 2,
            [pltpu.VMEM((CHUNK,), jnp.int32)] * 2,
            [pltpu.VMEM((CHUNK,), jnp.int32)] * 2,
            [pltpu.VMEM((CHUNK,), jnp.int32)] * 2,
            [pltpu.VMEM((CHUNK,), jnp.float32)] * 2,
            [pltpu.VMEM((CHUNK,), jnp.float32)] * 2,
            pltpu.VMEM((L,), jnp.float32),
            pltpu.VMEM_SHARED((NPAD, HIDDEN), jnp.float32),
            pltpu.VMEM_SHARED((N,), jnp.float32),
            pltpu.VMEM_SHARED((N,), jnp.float32),
            [pltpu.SemaphoreType.DMA] * 2,
            [pltpu.SemaphoreType.DMA] * 2,
            [pltpu.SemaphoreType.DMA] * 2,
            [pltpu.SemaphoreType.DMA] * 2,
        ],
    )
    return fn(h, es, ed, m, src, dst)


def _post_body(acc_ref, den_ref, wout_ref, out_ref):
    a = acc_ref[0, :N, :] + acc_ref[1, :N, :]
    d = jnp.sum(den_ref[:, 0, :], axis=0)
    x = a / (d[:, None] + 1e-16)
    y = jnp.where(x > 0.0, x, jnp.exp(jnp.minimum(x, 0.0)) - 1.0)
    out_ref[...] = lax.dot_general(
        y, wout_ref[...], (((1,), (1,)), ((), ())),
        preferred_element_type=jnp.float32)


def _tc_post(accp, denp, W_out):
    return pl.pallas_call(
        _post_body,
        out_shape=jax.ShapeDtypeStruct((N, HIDDEN), jnp.float32),
    )(accp, denp, W_out)


def kernel(feat, edge_index, W, a_src, a_dst, W_out):
    h, es, ed, m = _tc_pre(feat, W, a_src, a_dst)
    src = edge_index[0]
    dst = edge_index[1]
    m16 = jnp.broadcast_to(m.reshape(1), (16,))
    accp, denp = _sc_agg(h, es.reshape(N), ed.reshape(N), m16, src, dst)
    return _tc_post(accp, denp, W_out)


# speedup vs baseline: 1.0558x; 1.0558x over previous
import functools

import jax
import jax.numpy as jnp
from jax import lax
from jax.experimental import pallas as pl
from jax.experimental.pallas import tpu as pltpu
from jax.experimental.pallas import tpu_sc as plsc

N = 10000
E = 320000
HIDDEN = 128

NC = 2
NS = 16
L = 16
NW = NC * NS
EPW = E // NW
CHUNK = 80
HA = 48
HB = CHUNK - HA
NCHUNK = EPW // CHUNK
NPAD = 10240
RPT = NPAD // NS


def _pre_body(feat_ref, w_ref, asrc_ref, adst_ref, h_ref, es_ref, ed_ref, m_ref):
    h = jnp.dot(feat_ref[...], w_ref[...], preferred_element_type=jnp.float32)
    h_ref[...] = h
    es = jnp.dot(h, asrc_ref[0, :], preferred_element_type=jnp.float32)
    ed = jnp.dot(h, adst_ref[0, :], preferred_element_type=jnp.float32)
    es_ref[0, :] = es
    ed_ref[0, :] = ed
    t = jnp.max(es) + jnp.max(ed)
    m_ref[0, 0] = jnp.where(t >= 0.0, t, 0.2 * t)


def _tc_pre(feat, W, a_src, a_dst):
    return pl.pallas_call(
        _pre_body,
        out_shape=(
            jax.ShapeDtypeStruct((N, HIDDEN), jnp.float32),
            jax.ShapeDtypeStruct((1, N), jnp.float32),
            jax.ShapeDtypeStruct((1, N), jnp.float32),
            jax.ShapeDtypeStruct((1, 1), jnp.float32),
        ),
        out_specs=(
            pl.BlockSpec((N, HIDDEN), lambda: (0, 0)),
            pl.BlockSpec((1, N), lambda: (0, 0)),
            pl.BlockSpec((1, N), lambda: (0, 0)),
            pl.BlockSpec(memory_space=pltpu.SMEM),
        ),
    )(feat, W, a_src.reshape(1, HIDDEN), a_dst.reshape(1, HIDDEN))


def _sc_body(h_hbm, es_hbm, ed_hbm, m_hbm, src_hbm, dst_hbm,
             acc_out, den_out,
             den_v, rows_v, srcb, dstb, dstA, dstB, esb, edb, m_v,
             acc_sh, es_sp, ed_sp,
             sem_idx, sem_rows, sem_e, sem_scat):
    cid = lax.axis_index("c")
    sid = lax.axis_index("s")
    wid = sid * NC + cid
    base = wid * EPW

    pltpu.sync_copy(m_hbm, m_v)

    @pl.when(sid == 0)
    def _():
        pltpu.sync_copy(es_hbm, es_sp)
        pltpu.sync_copy(ed_hbm, ed_sp)

    def _zrow(j, _):
        for k in range(HIDDEN // L):
            rows_v[0][j, pl.ds(k * L, L)] = jnp.zeros((L,), jnp.float32)
        return 0
    lax.fori_loop(0, CHUNK, _zrow, 0)

    def _zden(j, _):
        den_v[pl.ds(j * L, L)] = jnp.zeros((L,), jnp.float32)
        return 0
    lax.fori_loop(0, N // L, _zden, 0)

    for t in range(RPT // CHUNK):
        pltpu.sync_copy(rows_v[0], acc_sh.at[pl.ds(sid * RPT + t * CHUNK, CHUNK)])
    plsc.subcore_barrier()

    def _copy_idx(i, b):
        off = base + i * CHUNK
        pltpu.async_copy(src_hbm.at[pl.ds(off, CHUNK)], srcb[b], sem_idx[b])
        pltpu.async_copy(dst_hbm.at[pl.ds(off, CHUNK)], dstb[b], sem_idx[b])

    def _wait_idx(b):
        pltpu.make_async_copy(src_hbm.at[pl.ds(0, CHUNK)], srcb[b], sem_idx[b]).wait()
        pltpu.make_async_copy(dst_hbm.at[pl.ds(0, CHUNK)], dstb[b], sem_idx[b]).wait()

    def _start_fetch(b):
        _wait_idx(b)
        pltpu.async_copy(h_hbm.at[srcb[b]], rows_v[b], sem_rows[b])
        pltpu.async_copy(es_sp.at[srcb[b]], esb[b], sem_e[b])
        pltpu.async_copy(ed_sp.at[dstb[b]], edb[b], sem_e[b])

    def _wait_fetch(b):
        pltpu.make_async_copy(h_hbm.at[srcb[b]], rows_v[b], sem_rows[b]).wait()
        pltpu.make_async_copy(es_sp.at[srcb[b]], esb[b], sem_e[b]).wait()
        pltpu.make_async_copy(ed_sp.at[dstb[b]], edb[b], sem_e[b]).wait()

    def _wait_scat(b):
        pltpu.make_async_copy(rows_v[b].at[pl.ds(0, HA)],
                              acc_sh.at[dstA[b]], sem_scat[b]).wait()
        pltpu.make_async_copy(rows_v[b].at[pl.ds(HA, HB)],
                              acc_sh.at[dstB[b]], sem_scat[b]).wait()

    def _snap_dst(b):
        for g in range(HA // L):
            dstA[b][pl.ds(g * L, L)] = dstb[b][pl.ds(g * L, L)]
        for g in range(HB // L):
            dstB[b][pl.ds(g * L, L)] = dstb[b][pl.ds(HA + g * L, L)]

    def _compute_span(b, g0, g1):
        for g in range(g0, g1):
            idx = dstA[b] if g < HA // L else dstB[b]
            goff = g * L if g < HA // L else g * L - HA
            s = esb[b][pl.ds(g * L, L)] + edb[b][pl.ds(g * L, L)]
            e = jnp.where(s >= 0.0, s, 0.2 * s)
            au = jnp.exp(e - m_v[...])
            plsc.addupdate_scatter(den_v, [idx[pl.ds(goff, L)]], au)
            for jl in range(L):
                ab = jnp.take_along_axis(au, jnp.full((L,), jl, jnp.int32), axis=0)
                j = g * L + jl
                for k in range(HIDDEN // L):
                    rows_v[b][j, pl.ds(k * L, L)] = rows_v[b][j, pl.ds(k * L, L)] * ab

    _copy_idx(0, 0)
    _start_fetch(0)
    _copy_idx(1, 1)

    def _step(i, b, first, last):
        _wait_fetch(b)
        _snap_dst(b)
        if not last:
            @pl.when(i + 2 < NCHUNK)
            def _():
                _copy_idx(i + 2, b)
        if first:
            @pl.when(i >= 1)
            def _():
                _wait_scat(1 - b)
        else:
            _wait_scat(1 - b)
        if not last:
            _start_fetch(1 - b)
        _compute_span(b, 0, HA // L)
        pltpu.async_copy(rows_v[b].at[pl.ds(0, HA)], acc_sh.at[dstA[b]],
                         add=True, sem=sem_scat[b])
        _compute_span(b, HA // L, CHUNK // L)
        pltpu.async_copy(rows_v[b].at[pl.ds(HA, HB)], acc_sh.at[dstB[b]],
                         add=True, sem=sem_scat[b])

    def _pair(k, _):
        _step(2 * k, 0, first=True, last=False)
        _step(2 * k + 1, 1, first=False, last=False)
        return 0

    lax.fori_loop(0, (NCHUNK - 1) // 2, _pair, 0)
    _step(NCHUNK - 1, (NCHUNK - 1) % 2, first=False, last=True)
    _wait_scat((NCHUNK - 1) % 2)

    pltpu.sync_copy(den_v, den_out.at[wid, 0])
    plsc.subcore_barrier()
    pltpu.sync_copy(acc_sh.at[pl.ds(sid * RPT, RPT)],
                    acc_out.at[cid, pl.ds(sid * RPT, RPT)])


def _sc_agg(h, es, ed, m, src, dst):
    fn = pl.kernel(
        _sc_body,
        mesh=plsc.VectorSubcoreMesh(core_axis_name="c", subcore_axis_name="s"),
        compiler_params=pltpu.CompilerParams(needs_layout_passes=False),
        out_type=(
            jax.ShapeDtypeStruct((NC, NPAD, HIDDEN), jnp.float32),
            jax.ShapeDtypeStruct((NW, 1, N), jnp.float32),
        ),
        scratch_types=[
            pltpu.VMEM((N,), jnp.float32),
            [pltpu.VMEM((CHUNK, HIDDEN), jnp.float32)] * 2,
            [pltpu.VMEM((CHUNK,), jnp.int32)] * 2,
            [pltpu.VMEM((CHUNK,), jnp.int32)] * 2,
            [pltpu.VMEM((HA,), jnp.int32)] * 2,
            [pltpu.VMEM((HB,), jnp.int32)] * 2,
            [pltpu.VMEM((CHUNK,), jnp.float32)] * 2,
            [pltpu.VMEM((CHUNK,), jnp.float32)] * 2,
            pltpu.VMEM((L,), jnp.float32),
            pltpu.VMEM_SHARED((NPAD, HIDDEN), jnp.float32),
            pltpu.VMEM_SHARED((N,), jnp.float32),
            pltpu.VMEM_SHARED((N,), jnp.float32),
            [pltpu.SemaphoreType.DMA] * 2,
            [pltpu.SemaphoreType.DMA] * 2,
            [pltpu.SemaphoreType.DMA] * 2,
            [pltpu.SemaphoreType.DMA] * 2,
        ],
    )
    return fn(h, es, ed, m, src, dst)


def _post_body(acc_ref, den_ref, wout_ref, out_ref):
    a = acc_ref[0, :N, :] + acc_ref[1, :N, :]
    d = jnp.sum(den_ref[:, 0, :], axis=0)
    x = a / (d[:, None] + 1e-16)
    y = jnp.where(x > 0.0, x, jnp.exp(jnp.minimum(x, 0.0)) - 1.0)
    out_ref[...] = lax.dot_general(
        y, wout_ref[...], (((1,), (1,)), ((), ())),
        preferred_element_type=jnp.float32)


def _tc_post(accp, denp, W_out):
    return pl.pallas_call(
        _post_body,
        out_shape=jax.ShapeDtypeStruct((N, HIDDEN), jnp.float32),
    )(accp, denp, W_out)


def kernel(feat, edge_index, W, a_src, a_dst, W_out):
    h, es, ed, m = _tc_pre(feat, W, a_src, a_dst)
    src = edge_index[0]
    dst = edge_index[1]
    m16 = jnp.broadcast_to(m.reshape(1), (16,))
    accp, denp = _sc_agg(h, es.reshape(N), ed.reshape(N), m16, src, dst)
    return _tc_post(accp, denp, W_out)
